# Initial kernel scaffold; baseline (speedup 1.0000x reference)
import jax
import jax.numpy as jnp
from jax.experimental import pallas as pl


def kernel(x, edge_index, edge_attr, batch, W_enc, b_enc, eps, We, be, W1, b1, g1, bt1, W2, b2, g2, bt2, Wp1, bp1, Wp2, bp2, Wpb, bpb, Wpo, bpo):
    raise NotImplementedError("write your pallas kernel here")


# speedup vs baseline: 1.5833x; 1.5833x over previous
import functools

import jax
import jax.numpy as jnp
from jax import lax
from jax.experimental import pallas as pl
from jax.experimental.pallas import tpu as pltpu
from jax.experimental.pallas import tpu_sc as plsc

N = 10000
E = 160000
F_IN = 256
F_EDGE = 16
H = 512
HC = 128
NCH = H // HC
G = 64
PH = 2048
PB = 256
PO = 256

RB = 2000
NRB = N // RB
EB = 8000
NEB = E // EB

SC_TILES = 16
EPT = E // SC_TILES
KB = 80
NBATCH = EPT // KB
NP = 10240
RPT = NP // SC_TILES
ZR = 128


def _sc_msg_body(h_ref, e_ref, src_ref, dst_ref, out_ref,
                 agg_s, idx_v, didx_v, rows_v, e_v, zer_v, sem):
    cid = lax.axis_index("c")
    sid = lax.axis_index("s")

    zv = jnp.zeros((16,), jnp.float32)

    def zfill(i, _):
        for j in range(HC // 16):
            zer_v[i, pl.ds(j * 16, 16)] = zv
        return 0

    lax.fori_loop(0, ZR, zfill, 0)

    for c in range(NCH):
        @pl.when(cid == c // 2)
        def _(c=c):
            for j in range(RPT // ZR):
                pltpu.sync_copy(zer_v, agg_s.at[pl.ds(sid * RPT + j * ZR, ZR)])
            plsc.subcore_barrier()

            def edge_batch(bi, _):
                b = sid * EPT + bi * KB
                pltpu.sync_copy(src_ref.at[pl.ds(b, KB)], idx_v)
                pltpu.sync_copy(dst_ref.at[pl.ds(b, KB)], didx_v)
                if c > 0:
                    for j in range(KB // 16):
                        sl = pl.ds(j * 16, 16)
                        idx_v[sl] = idx_v[sl] + (c * N)
                pltpu.async_copy(h_ref.at[idx_v], rows_v, sem).wait()
                pltpu.sync_copy(e_ref.at[pl.ds(c * E + b, KB)], e_v)

                def row_body(r, _):
                    for j in range(HC // 16):
                        sl = pl.ds(j * 16, 16)
                        rows_v[r, sl] = jnp.maximum(rows_v[r, sl] + e_v[r, sl],
                                                    0.0)
                    return 0

                lax.fori_loop(0, KB, row_body, 0)
                pltpu.sync_copy(rows_v, agg_s.at[didx_v], add=True)
                return 0

            lax.fori_loop(0, NBATCH, edge_batch, 0)
            plsc.subcore_barrier()
            for j in range(RPT // ZR):
                r0 = sid * RPT + j * ZR
                pltpu.sync_copy(agg_s.at[pl.ds(r0, ZR)],
                                out_ref.at[pl.ds(c * NP + r0, ZR)])
            plsc.subcore_barrier()


def _sc_message_agg(h_flat, e_flat, src, dst):
    mesh = plsc.VectorSubcoreMesh(core_axis_name="c", subcore_axis_name="s")
    return pl.kernel(
        _sc_msg_body,
        out_type=jax.ShapeDtypeStruct((NCH * NP, HC), jnp.float32),
        mesh=mesh,
        scratch_types=[
            pltpu.MemorySpace.VMEM_SHARED((NP, HC), jnp.float32),
            pltpu.VMEM((KB,), jnp.int32),
            pltpu.VMEM((KB,), jnp.int32),
            pltpu.VMEM((KB, HC), jnp.float32),
            pltpu.VMEM((KB, HC), jnp.float32),
            pltpu.VMEM((ZR, HC), jnp.float32),
            pltpu.SemaphoreType.DMA,
        ],
    )(h_flat, e_flat, src, dst)


def _enc_body(x_ref, w_ref, b_ref, out_ref):
    out_ref[0] = jnp.dot(x_ref[...], w_ref[...],
                         preferred_element_type=jnp.float32) + b_ref[...]


def _encoder(x, W_enc, b_enc):
    return pl.pallas_call(
        _enc_body,
        grid=(NCH, NRB),
        in_specs=[
            pl.BlockSpec((RB, F_IN), lambda c, r: (r, 0)),
            pl.BlockSpec((F_IN, HC), lambda c, r: (0, c)),
            pl.BlockSpec((1, HC), lambda c, r: (0, c)),
        ],
        out_specs=pl.BlockSpec((1, RB, HC), lambda c, r: (c, r, 0)),
        out_shape=jax.ShapeDtypeStruct((NCH, N, HC), jnp.float32),
    )(x, W_enc, b_enc.reshape(1, H))


def _edge_body(ea_ref, w_ref, b_ref, out_ref):
    out_ref[0] = jnp.dot(ea_ref[...], w_ref[...],
                         preferred_element_type=jnp.float32) + b_ref[...]


def _edge_mlp(edge_attr, We_l, be_l):
    return pl.pallas_call(
        _edge_body,
        grid=(NCH, NEB),
        in_specs=[
            pl.BlockSpec((EB, F_EDGE), lambda c, r: (r, 0)),
            pl.BlockSpec((F_EDGE, HC), lambda c, r: (0, c)),
            pl.BlockSpec((1, HC), lambda c, r: (0, c)),
        ],
        out_specs=pl.BlockSpec((1, EB, HC), lambda c, r: (c, r, 0)),
        out_shape=jax.ShapeDtypeStruct((NCH, E, HC), jnp.float32),
    )(edge_attr, We_l, be_l.reshape(1, H))


def _k1_body(h4_ref, a4_ref, eps_ref, w_ref, b_ref, y_ref, st_ref):
    r = pl.program_id(0)
    hcat = jnp.concatenate([h4_ref[c] for c in range(NCH)], axis=1)
    acat = jnp.concatenate([a4_ref[c] for c in range(NCH)], axis=1)
    z = (1.0 + eps_ref[0, 0]) * hcat + acat
    y = jnp.dot(z, w_ref[...], preferred_element_type=jnp.float32) + b_ref[...]
    y_ref[...] = y
    cs = jnp.sum(y, axis=0, keepdims=True)
    cq = jnp.sum(y * y, axis=0, keepdims=True)
    row = lax.broadcasted_iota(jnp.int32, (8, 2 * H), 0)
    upd = jnp.where(row == 0, cs, 0.0) + jnp.where(row == 1, cq, 0.0)

    @pl.when(r == 0)
    def _():
        st_ref[...] = jnp.zeros_like(st_ref)

    st_ref[...] += upd


def _k1(H4, A4, eps_l, W1_l, b1_l):
    return pl.pallas_call(
        _k1_body,
        grid=(NRB,),
        in_specs=[
            pl.BlockSpec((NCH, RB, HC), lambda r: (0, r, 0)),
            pl.BlockSpec((NCH, RB, HC), lambda r: (0, r, 0)),
            pl.BlockSpec(memory_space=pltpu.MemorySpace.SMEM),
            pl.BlockSpec((H, 2 * H), lambda r: (0, 0)),
            pl.BlockSpec((1, 2 * H), lambda r: (0, 0)),
        ],
        out_specs=[
            pl.BlockSpec((RB, 2 * H), lambda r: (r, 0)),
            pl.BlockSpec((8, 2 * H), lambda r: (0, 0)),
        ],
        out_shape=[
            jax.ShapeDtypeStruct((N, 2 * H), jnp.float32),
            jax.ShapeDtypeStruct((8, 2 * H), jnp.float32),
        ],
    )(H4, A4, eps_l.reshape(1, 1), W1_l, b1_l.reshape(1, 2 * H))


def _bn_coeffs(st, g, bt):
    mean = st[0:1, :] / N
    var = st[1:2, :] / N - mean * mean
    a = g * lax.rsqrt(var + 1e-5)
    b = bt - mean * a
    return a, b


def _k2_body(y_ref, st_ref, g_ref, bt_ref, w_ref, b_ref, y2_ref, st2_ref):
    r = pl.program_id(0)
    a, b = _bn_coeffs(st_ref[...], g_ref[...], bt_ref[...])
    t = jnp.maximum(y_ref[...] * a + b, 0.0)
    y2 = jnp.dot(t, w_ref[...], preferred_element_type=jnp.float32) + b_ref[...]
    y2_ref[...] = y2
    cs = jnp.sum(y2, axis=0, keepdims=True)
    cq = jnp.sum(y2 * y2, axis=0, keepdims=True)
    row = lax.broadcasted_iota(jnp.int32, (8, H), 0)
    upd = jnp.where(row == 0, cs, 0.0) + jnp.where(row == 1, cq, 0.0)

    @pl.when(r == 0)
    def _():
        st2_ref[...] = jnp.zeros_like(st2_ref)

    st2_ref[...] += upd


def _k2(y, st1, g1_l, bt1_l, W2_l, b2_l):
    return pl.pallas_call(
        _k2_body,
        grid=(NRB,),
        in_specs=[
            pl.BlockSpec((RB, 2 * H), lambda r: (r, 0)),
            pl.BlockSpec((8, 2 * H), lambda r: (0, 0)),
            pl.BlockSpec((1, 2 * H), lambda r: (0, 0)),
            pl.BlockSpec((1, 2 * H), lambda r: (0, 0)),
            pl.BlockSpec((2 * H, H), lambda r: (0, 0)),
            pl.BlockSpec((1, H), lambda r: (0, 0)),
        ],
        out_specs=[
            pl.BlockSpec((RB, H), lambda r: (r, 0)),
            pl.BlockSpec((8, H), lambda r: (0, 0)),
        ],
        out_shape=[
            jax.ShapeDtypeStruct((N, H), jnp.float32),
            jax.ShapeDtypeStruct((8, H), jnp.float32),
        ],
    )(y, st1, g1_l.reshape(1, 2 * H), bt1_l.reshape(1, 2 * H), W2_l,
      b2_l.reshape(1, H))


def _k3_body(y2_ref, st_ref, g_ref, bt_ref, h4_ref):
    a, b = _bn_coeffs(st_ref[...], g_ref[...], bt_ref[...])
    h4_ref[0] = jnp.maximum(y2_ref[...] * a + b, 0.0)


def _k3(y2, st2, g2_l, bt2_l):
    return pl.pallas_call(
        _k3_body,
        grid=(NCH, NRB),
        in_specs=[
            pl.BlockSpec((RB, HC), lambda c, r: (r, c)),
            pl.BlockSpec((8, HC), lambda c, r: (0, c)),
            pl.BlockSpec((1, HC), lambda c, r: (0, c)),
            pl.BlockSpec((1, HC), lambda c, r: (0, c)),
        ],
        out_specs=pl.BlockSpec((1, RB, HC), lambda c, r: (c, r, 0)),
        out_shape=jax.ShapeDtypeStruct((NCH, N, HC), jnp.float32),
    )(y2, st2, g2_l.reshape(1, H), bt2_l.reshape(1, H))


def _k3pool_body(y2_ref, st_ref, g_ref, bt_ref, bf_ref, sums_ref, cnt_ref):
    c = pl.program_id(0)
    r = pl.program_id(1)
    a, b = _bn_coeffs(st_ref[...], g_ref[...], bt_ref[...])
    h = jnp.maximum(y2_ref[...] * a + b, 0.0)
    gid = lax.broadcasted_iota(jnp.int32, (RB, G), 1)
    onehot = (bf_ref[...] == gid).astype(jnp.float32)
    part = lax.dot_general(onehot, h, (((0,), (0,)), ((), ())),
                           preferred_element_type=jnp.float32)

    @pl.when(r == 0)
    def _():
        sums_ref[...] = jnp.zeros_like(sums_ref)

    sums_ref[...] += part

    @pl.when(c == 0)
    def _():
        cpart = lax.dot_general(onehot, jnp.ones((RB, HC), jnp.float32),
                                (((0,), (0,)), ((), ())),
                                preferred_element_type=jnp.float32)

        @pl.when(r == 0)
        def _():
            cnt_ref[...] = jnp.zeros_like(cnt_ref)

        cnt_ref[...] += cpart


def _k3pool(y2, st2, g2_l, bt2_l, batch_f):
    return pl.pallas_call(
        _k3pool_body,
        grid=(NCH, NRB),
        in_specs=[
            pl.BlockSpec((RB, HC), lambda c, r: (r, c)),
            pl.BlockSpec((8, HC), lambda c, r: (0, c)),
            pl.BlockSpec((1, HC), lambda c, r: (0, c)),
            pl.BlockSpec((1, HC), lambda c, r: (0, c)),
            pl.BlockSpec((RB, 1), lambda c, r: (r, 0)),
        ],
        out_specs=[
            pl.BlockSpec((G, HC), lambda c, r: (0, c)),
            pl.BlockSpec((G, HC), lambda c, r: (0, 0)),
        ],
        out_shape=[
            jax.ShapeDtypeStruct((G, H), jnp.float32),
            jax.ShapeDtypeStruct((G, HC), jnp.float32),
        ],
    )(y2, st2, g2_l.reshape(1, H), bt2_l.reshape(1, H), batch_f)


def _gelu(x):
    return 0.5 * x * (1.0 + lax.erf(x * (2.0 ** -0.5)))


def _proj_body(sums_ref, cnt_ref, w1_ref, b1_ref, w2_ref, b2_ref, wb_ref,
               bb_ref, wo_ref, bo_ref, out_ref):
    emb = sums_ref[...] / jnp.maximum(cnt_ref[:, 0:1], 1.0)
    p = _gelu(jnp.dot(emb, w1_ref[...], preferred_element_type=jnp.float32)
              + b1_ref[...])
    p = _gelu(jnp.dot(p, w2_ref[...], preferred_element_type=jnp.float32)
              + b2_ref[...])
    p = _gelu(jnp.dot(p, wb_ref[...], preferred_element_type=jnp.float32)
              + bb_ref[...])
    p = jnp.dot(p, wo_ref[...], preferred_element_type=jnp.float32) + bo_ref[...]
    nrm = jnp.maximum(
        jnp.sqrt(jnp.sum(p * p, axis=1, keepdims=True)), 1e-12)
    out_ref[...] = p / nrm


def _projection(sums, cnt, Wp1, bp1, Wp2, bp2, Wpb, bpb, Wpo, bpo):
    return pl.pallas_call(
        _proj_body,
        out_shape=jax.ShapeDtypeStruct((G, PO), jnp.float32),
    )(sums, cnt, Wp1, bp1.reshape(1, PH), Wp2, bp2.reshape(1, PH),
      Wpb, bpb.reshape(1, PB), Wpo, bpo.reshape(1, PO))


def kernel(x, edge_index, edge_attr, batch, W_enc, b_enc, eps, We, be, W1, b1,
           g1, bt1, W2, b2, g2, bt2, Wp1, bp1, Wp2, bp2, Wpb, bpb, Wpo, bpo):
    src = edge_index[0]
    dst = edge_index[1]
    batch_i = batch.reshape(N, 1)

    H4 = _encoder(x, W_enc, b_enc)
    for l in range(3):
        e4 = _edge_mlp(edge_attr, We[l], be[l])
        agg = _sc_message_agg(H4.reshape(NCH * N, HC),
                              e4.reshape(NCH * E, HC), src, dst)
        y, st1 = _k1(H4, agg.reshape(NCH, NP, HC), eps[l], W1[l], b1[l])
        y2, st2 = _k2(y, st1, g1[l], bt1[l], W2[l], b2[l])
        if l < 2:
            H4 = _k3(y2, st2, g2[l], bt2[l])
        else:
            sums, cnt = _k3pool(y2, st2, g2[l], bt2[l], batch_i)
    return _projection(sums, cnt, Wp1, bp1, Wp2, bp2, Wpb, bpb, Wpo, bpo)


# speedup vs baseline: 2.4631x; 1.5557x over previous
import functools

import jax
import jax.numpy as jnp
from jax import lax
from jax.experimental import pallas as pl
from jax.experimental.pallas import tpu as pltpu
from jax.experimental.pallas import tpu_sc as plsc

N = 10000
E = 160000
F_IN = 256
F_EDGE = 16
H = 512
HC = 128
NCH = H // HC
G = 64
PH = 2048
PB = 256
PO = 256

RB = 2000
NRB = N // RB
EB = 8000
NEB = E // EB

SC_TILES = 16
EPT = E // SC_TILES
KB = 40
KBP = 48
NBATCH = EPT // KB
NP = 10240
RPT = NP // SC_TILES


def _sc_msg_body(h_ref, e_ref, src_ref, dst_ref, out_ref,
                 agg_s, sA, sB, dA, dB, rA, rB, eA, eB,
                 semSA, semSB, semDEA, semDEB, semGA, semGB):
    cid = lax.axis_index("c")
    sid = lax.axis_index("s")

    def relu_add(rbuf, ebuf):
        @plsc.parallel_loop(0, KB, 1, unroll=2)
        def _(r):
            for j in range(HC // 16):
                sl = pl.ds(j * 16, 16)
                rbuf[r, sl] = jnp.maximum(rbuf[r, sl] + ebuf[r, sl], 0.0)

    zv = jnp.zeros((16,), jnp.float32)

    for c in range(NCH):
        @pl.when(cid == c // 2)
        def _(c=c):
            base = sid * EPT
            ebase = c * E + base

            def zfill(i, _):
                for j in range(HC // 16):
                    rA[i, pl.ds(j * 16, 16)] = zv
                return 0

            lax.fori_loop(0, KB, zfill, 0)
            for j in range(RPT // KB):
                pltpu.sync_copy(rA, agg_s.at[pl.ds(sid * RPT + j * KB, KB)])
            plsc.subcore_barrier()

            def start(b, sbuf, dbuf, ebuf, semS, semDE):
                pltpu.async_copy(src_ref.at[pl.ds(base + b * KB, KB)],
                                 sbuf.at[pl.ds(0, KB)], semS)
                pltpu.async_copy(dst_ref.at[pl.ds(base + b * KB, KB)], dbuf,
                                 semDE)
                pltpu.async_copy(e_ref.at[pl.ds(ebase + b * KB, KB)], ebuf,
                                 semDE)

            def mid(b, sbuf, rbuf, semS, semG):
                pltpu.make_async_copy(src_ref.at[pl.ds(base + b * KB, KB)],
                                      sbuf.at[pl.ds(0, KB)], semS).wait()
                if c > 0:
                    for j in range(KBP // 16):
                        sl = pl.ds(j * 16, 16)
                        sbuf[sl] = sbuf[sl] + (c * N)
                pltpu.async_copy(h_ref.at[sbuf.at[pl.ds(0, KB)]], rbuf, semG)

            def finish(b, sbuf, dbuf, rbuf, ebuf, semDE, semG):
                pltpu.make_async_copy(dst_ref.at[pl.ds(base + b * KB, KB)],
                                      dbuf, semDE).wait()
                pltpu.make_async_copy(e_ref.at[pl.ds(ebase + b * KB, KB)],
                                      ebuf, semDE).wait()
                pltpu.make_async_copy(h_ref.at[sbuf.at[pl.ds(0, KB)]], rbuf,
                                      semG).wait()
                relu_add(rbuf, ebuf)
                pltpu.sync_copy(rbuf, agg_s.at[dbuf], add=True)

            start(0, sA, dA, eA, semSA, semDEA)
            mid(0, sA, rA, semSA, semGA)

            def pair(i, _):
                b0 = 2 * i
                start(b0 + 1, sB, dB, eB, semSB, semDEB)
                mid(b0 + 1, sB, rB, semSB, semGB)
                finish(b0, sA, dA, rA, eA, semDEA, semGA)
                start(b0 + 2, sA, dA, eA, semSA, semDEA)
                mid(b0 + 2, sA, rA, semSA, semGA)
                finish(b0 + 1, sB, dB, rB, eB, semDEB, semGB)
                return 0

            lax.fori_loop(0, NBATCH // 2 - 1, pair, 0)
            start(NBATCH - 1, sB, dB, eB, semSB, semDEB)
            mid(NBATCH - 1, sB, rB, semSB, semGB)
            finish(NBATCH - 2, sA, dA, rA, eA, semDEA, semGA)
            finish(NBATCH - 1, sB, dB, rB, eB, semDEB, semGB)

            plsc.subcore_barrier()
            for j in range(RPT // KB):
                r0 = sid * RPT + j * KB
                pltpu.sync_copy(agg_s.at[pl.ds(r0, KB)],
                                out_ref.at[pl.ds(c * NP + r0, KB)])
            plsc.subcore_barrier()


def _sc_message_agg(h_flat, e_flat, src, dst):
    mesh = plsc.VectorSubcoreMesh(core_axis_name="c", subcore_axis_name="s")
    return pl.kernel(
        _sc_msg_body,
        out_type=jax.ShapeDtypeStruct((NCH * NP, HC), jnp.float32),
        mesh=mesh,
        scratch_types=[
            pltpu.MemorySpace.VMEM_SHARED((NP, HC), jnp.float32),
            pltpu.VMEM((KBP,), jnp.int32),
            pltpu.VMEM((KBP,), jnp.int32),
            pltpu.VMEM((KB,), jnp.int32),
            pltpu.VMEM((KB,), jnp.int32),
            pltpu.VMEM((KB, HC), jnp.float32),
            pltpu.VMEM((KB, HC), jnp.float32),
            pltpu.VMEM((KB, HC), jnp.float32),
            pltpu.VMEM((KB, HC), jnp.float32),
            pltpu.SemaphoreType.DMA,
            pltpu.SemaphoreType.DMA,
            pltpu.SemaphoreType.DMA,
            pltpu.SemaphoreType.DMA,
            pltpu.SemaphoreType.DMA,
            pltpu.SemaphoreType.DMA,
        ],
    )(h_flat, e_flat, src, dst)


def _enc_body(x_ref, w_ref, b_ref, out_ref):
    out_ref[0] = jnp.dot(x_ref[...], w_ref[...],
                         preferred_element_type=jnp.float32) + b_ref[...]


def _encoder(x, W_enc, b_enc):
    return pl.pallas_call(
        _enc_body,
        grid=(NCH, NRB),
        in_specs=[
            pl.BlockSpec((RB, F_IN), lambda c, r: (r, 0)),
            pl.BlockSpec((F_IN, HC), lambda c, r: (0, c)),
            pl.BlockSpec((1, HC), lambda c, r: (0, c)),
        ],
        out_specs=pl.BlockSpec((1, RB, HC), lambda c, r: (c, r, 0)),
        out_shape=jax.ShapeDtypeStruct((NCH, N, HC), jnp.float32),
    )(x, W_enc, b_enc.reshape(1, H))


def _edge_body(ea_ref, w_ref, b_ref, out_ref):
    out_ref[0] = jnp.dot(ea_ref[...], w_ref[...],
                         preferred_element_type=jnp.float32) + b_ref[...]


def _edge_mlp(edge_attr, We_l, be_l):
    return pl.pallas_call(
        _edge_body,
        grid=(NCH, NEB),
        in_specs=[
            pl.BlockSpec((EB, F_EDGE), lambda c, r: (r, 0)),
            pl.BlockSpec((F_EDGE, HC), lambda c, r: (0, c)),
            pl.BlockSpec((1, HC), lambda c, r: (0, c)),
        ],
        out_specs=pl.BlockSpec((1, EB, HC), lambda c, r: (c, r, 0)),
        out_shape=jax.ShapeDtypeStruct((NCH, E, HC), jnp.float32),
    )(edge_attr, We_l, be_l.reshape(1, H))


def _k1_body(h4_ref, a4_ref, eps_ref, w_ref, b_ref, y_ref, st_ref):
    r = pl.program_id(0)
    hcat = jnp.concatenate([h4_ref[c] for c in range(NCH)], axis=1)
    acat = jnp.concatenate([a4_ref[c] for c in range(NCH)], axis=1)
    z = (1.0 + eps_ref[0, 0]) * hcat + acat
    y = jnp.dot(z, w_ref[...], preferred_element_type=jnp.float32) + b_ref[...]
    y_ref[...] = y
    cs = jnp.sum(y, axis=0, keepdims=True)
    cq = jnp.sum(y * y, axis=0, keepdims=True)
    row = lax.broadcasted_iota(jnp.int32, (8, 2 * H), 0)
    upd = jnp.where(row == 0, cs, 0.0) + jnp.where(row == 1, cq, 0.0)

    @pl.when(r == 0)
    def _():
        st_ref[...] = jnp.zeros_like(st_ref)

    st_ref[...] += upd


def _k1(H4, A4, eps_l, W1_l, b1_l):
    return pl.pallas_call(
        _k1_body,
        grid=(NRB,),
        in_specs=[
            pl.BlockSpec((NCH, RB, HC), lambda r: (0, r, 0)),
            pl.BlockSpec((NCH, RB, HC), lambda r: (0, r, 0)),
            pl.BlockSpec(memory_space=pltpu.MemorySpace.SMEM),
            pl.BlockSpec((H, 2 * H), lambda r: (0, 0)),
            pl.BlockSpec((1, 2 * H), lambda r: (0, 0)),
        ],
        out_specs=[
            pl.BlockSpec((RB, 2 * H), lambda r: (r, 0)),
            pl.BlockSpec((8, 2 * H), lambda r: (0, 0)),
        ],
        out_shape=[
            jax.ShapeDtypeStruct((N, 2 * H), jnp.float32),
            jax.ShapeDtypeStruct((8, 2 * H), jnp.float32),
        ],
    )(H4, A4, eps_l.reshape(1, 1), W1_l, b1_l.reshape(1, 2 * H))


def _bn_coeffs(st, g, bt):
    mean = st[0:1, :] / N
    var = st[1:2, :] / N - mean * mean
    a = g * lax.rsqrt(var + 1e-5)
    b = bt - mean * a
    return a, b


def _k2_body(y_ref, st_ref, g_ref, bt_ref, w_ref, b_ref, y2_ref, st2_ref):
    r = pl.program_id(0)
    a, b = _bn_coeffs(st_ref[...], g_ref[...], bt_ref[...])
    t = jnp.maximum(y_ref[...] * a + b, 0.0)
    y2 = jnp.dot(t, w_ref[...], preferred_element_type=jnp.float32) + b_ref[...]
    y2_ref[...] = y2
    cs = jnp.sum(y2, axis=0, keepdims=True)
    cq = jnp.sum(y2 * y2, axis=0, keepdims=True)
    row = lax.broadcasted_iota(jnp.int32, (8, H), 0)
    upd = jnp.where(row == 0, cs, 0.0) + jnp.where(row == 1, cq, 0.0)

    @pl.when(r == 0)
    def _():
        st2_ref[...] = jnp.zeros_like(st2_ref)

    st2_ref[...] += upd


def _k2(y, st1, g1_l, bt1_l, W2_l, b2_l):
    return pl.pallas_call(
        _k2_body,
        grid=(NRB,),
        in_specs=[
            pl.BlockSpec((RB, 2 * H), lambda r: (r, 0)),
            pl.BlockSpec((8, 2 * H), lambda r: (0, 0)),
            pl.BlockSpec((1, 2 * H), lambda r: (0, 0)),
            pl.BlockSpec((1, 2 * H), lambda r: (0, 0)),
            pl.BlockSpec((2 * H, H), lambda r: (0, 0)),
            pl.BlockSpec((1, H), lambda r: (0, 0)),
        ],
        out_specs=[
            pl.BlockSpec((RB, H), lambda r: (r, 0)),
            pl.BlockSpec((8, H), lambda r: (0, 0)),
        ],
        out_shape=[
            jax.ShapeDtypeStruct((N, H), jnp.float32),
            jax.ShapeDtypeStruct((8, H), jnp.float32),
        ],
    )(y, st1, g1_l.reshape(1, 2 * H), bt1_l.reshape(1, 2 * H), W2_l,
      b2_l.reshape(1, H))


def _k3_body(y2_ref, st_ref, g_ref, bt_ref, h4_ref):
    a, b = _bn_coeffs(st_ref[...], g_ref[...], bt_ref[...])
    h4_ref[0] = jnp.maximum(y2_ref[...] * a + b, 0.0)


def _k3(y2, st2, g2_l, bt2_l):
    return pl.pallas_call(
        _k3_body,
        grid=(NCH, NRB),
        in_specs=[
            pl.BlockSpec((RB, HC), lambda c, r: (r, c)),
            pl.BlockSpec((8, HC), lambda c, r: (0, c)),
            pl.BlockSpec((1, HC), lambda c, r: (0, c)),
            pl.BlockSpec((1, HC), lambda c, r: (0, c)),
        ],
        out_specs=pl.BlockSpec((1, RB, HC), lambda c, r: (c, r, 0)),
        out_shape=jax.ShapeDtypeStruct((NCH, N, HC), jnp.float32),
    )(y2, st2, g2_l.reshape(1, H), bt2_l.reshape(1, H))


def _k3pool_body(y2_ref, st_ref, g_ref, bt_ref, bf_ref, sums_ref, cnt_ref):
    c = pl.program_id(0)
    r = pl.program_id(1)
    a, b = _bn_coeffs(st_ref[...], g_ref[...], bt_ref[...])
    h = jnp.maximum(y2_ref[...] * a + b, 0.0)
    gid = lax.broadcasted_iota(jnp.int32, (RB, G), 1)
    onehot = (bf_ref[...] == gid).astype(jnp.float32)
    part = lax.dot_general(onehot, h, (((0,), (0,)), ((), ())),
                           preferred_element_type=jnp.float32)

    @pl.when(r == 0)
    def _():
        sums_ref[...] = jnp.zeros_like(sums_ref)

    sums_ref[...] += part

    @pl.when(c == 0)
    def _():
        cpart = lax.dot_general(onehot, jnp.ones((RB, HC), jnp.float32),
                                (((0,), (0,)), ((), ())),
                                preferred_element_type=jnp.float32)

        @pl.when(r == 0)
        def _():
            cnt_ref[...] = jnp.zeros_like(cnt_ref)

        cnt_ref[...] += cpart


def _k3pool(y2, st2, g2_l, bt2_l, batch_f):
    return pl.pallas_call(
        _k3pool_body,
        grid=(NCH, NRB),
        in_specs=[
            pl.BlockSpec((RB, HC), lambda c, r: (r, c)),
            pl.BlockSpec((8, HC), lambda c, r: (0, c)),
            pl.BlockSpec((1, HC), lambda c, r: (0, c)),
            pl.BlockSpec((1, HC), lambda c, r: (0, c)),
            pl.BlockSpec((RB, 1), lambda c, r: (r, 0)),
        ],
        out_specs=[
            pl.BlockSpec((G, HC), lambda c, r: (0, c)),
            pl.BlockSpec((G, HC), lambda c, r: (0, 0)),
        ],
        out_shape=[
            jax.ShapeDtypeStruct((G, H), jnp.float32),
            jax.ShapeDtypeStruct((G, HC), jnp.float32),
        ],
    )(y2, st2, g2_l.reshape(1, H), bt2_l.reshape(1, H), batch_f)


def _gelu(x):
    return 0.5 * x * (1.0 + lax.erf(x * (2.0 ** -0.5)))


def _proj_body(sums_ref, cnt_ref, w1_ref, b1_ref, w2_ref, b2_ref, wb_ref,
               bb_ref, wo_ref, bo_ref, out_ref):
    emb = sums_ref[...] / jnp.maximum(cnt_ref[:, 0:1], 1.0)
    p = _gelu(jnp.dot(emb, w1_ref[...], preferred_element_type=jnp.float32)
              + b1_ref[...])
    p = _gelu(jnp.dot(p, w2_ref[...], preferred_element_type=jnp.float32)
              + b2_ref[...])
    p = _gelu(jnp.dot(p, wb_ref[...], preferred_element_type=jnp.float32)
              + bb_ref[...])
    p = jnp.dot(p, wo_ref[...], preferred_element_type=jnp.float32) + bo_ref[...]
    nrm = jnp.maximum(
        jnp.sqrt(jnp.sum(p * p, axis=1, keepdims=True)), 1e-12)
    out_ref[...] = p / nrm


def _projection(sums, cnt, Wp1, bp1, Wp2, bp2, Wpb, bpb, Wpo, bpo):
    return pl.pallas_call(
        _proj_body,
        out_shape=jax.ShapeDtypeStruct((G, PO), jnp.float32),
    )(sums, cnt, Wp1, bp1.reshape(1, PH), Wp2, bp2.reshape(1, PH),
      Wpb, bpb.reshape(1, PB), Wpo, bpo.reshape(1, PO))


def kernel(x, edge_index, edge_attr, batch, W_enc, b_enc, eps, We, be, W1, b1,
           g1, bt1, W2, b2, g2, bt2, Wp1, bp1, Wp2, bp2, Wpb, bpb, Wpo, bpo):
    src = edge_index[0]
    dst = edge_index[1]
    batch_i = batch.reshape(N, 1)

    H4 = _encoder(x, W_enc, b_enc)
    for l in range(3):
        e4 = _edge_mlp(edge_attr, We[l], be[l])
        agg = _sc_message_agg(H4.reshape(NCH * N, HC),
                              e4.reshape(NCH * E, HC), src, dst)
        y, st1 = _k1(H4, agg.reshape(NCH, NP, HC), eps[l], W1[l], b1[l])
        y2, st2 = _k2(y, st1, g1[l], bt1[l], W2[l], b2[l])
        if l < 2:
            H4 = _k3(y2, st2, g2[l], bt2[l])
        else:
            sums, cnt = _k3pool(y2, st2, g2[l], bt2[l], batch_i)
    return _projection(sums, cnt, Wp1, bp1, Wp2, bp2, Wpb, bpb, Wpo, bpo)


# speedup vs baseline: 2.4652x; 1.0009x over previous
import functools

import jax
import jax.numpy as jnp
from jax import lax
from jax.experimental import pallas as pl
from jax.experimental.pallas import tpu as pltpu
from jax.experimental.pallas import tpu_sc as plsc

N = 10000
E = 160000
F_IN = 256
F_EDGE = 16
H = 512
HC = 128
NCH = H // HC
G = 64
PH = 2048
PB = 256
PO = 256

RB = 2000
NRB = N // RB
EB = 8000
NEB = E // EB

SC_TILES = 16
EPT = E // SC_TILES
KB = 40
KBP = 48
NBATCH = EPT // KB
NP = 10240
RPT = NP // SC_TILES


def _sc_msg_body(h_ref, e_ref, src_ref, dst_ref, out_ref,
                 agg_s, sA, sB, dA, dB, rA, rB, eA, eB,
                 semSA, semSB, semDEA, semDEB, semGA, semGB, semPA, semPB):
    cid = lax.axis_index("c")
    sid = lax.axis_index("s")

    def relu_add(rbuf, ebuf):
        @plsc.parallel_loop(0, KB, 1, unroll=2)
        def _(r):
            for j in range(HC // 16):
                sl = pl.ds(j * 16, 16)
                rbuf[r, sl] = jnp.maximum(rbuf[r, sl] + ebuf[r, sl], 0.0)

    zv = jnp.zeros((16,), jnp.float32)

    for c in range(NCH):
        @pl.when(cid == c // 2)
        def _(c=c):
            base = sid * EPT
            ebase = c * E + base

            def zfill(i, _):
                for j in range(HC // 16):
                    rA[i, pl.ds(j * 16, 16)] = zv
                return 0

            lax.fori_loop(0, KB, zfill, 0)
            for j in range(RPT // KB):
                pltpu.sync_copy(rA, agg_s.at[pl.ds(sid * RPT + j * KB, KB)])
            plsc.subcore_barrier()

            def start(b, sbuf, dbuf, rbuf, ebuf, semS, semDE, semP):
                @pl.when(b >= 2)
                def _():
                    pltpu.make_async_copy(rbuf, agg_s.at[dbuf], semP).wait()

                pltpu.async_copy(src_ref.at[pl.ds(base + b * KB, KB)],
                                 sbuf.at[pl.ds(0, KB)], semS)
                pltpu.async_copy(dst_ref.at[pl.ds(base + b * KB, KB)], dbuf,
                                 semDE)
                pltpu.async_copy(e_ref.at[pl.ds(ebase + b * KB, KB)], ebuf,
                                 semDE)

            def mid(b, sbuf, rbuf, semS, semG):
                pltpu.make_async_copy(src_ref.at[pl.ds(base + b * KB, KB)],
                                      sbuf.at[pl.ds(0, KB)], semS).wait()
                if c > 0:
                    for j in range(KBP // 16):
                        sl = pl.ds(j * 16, 16)
                        sbuf[sl] = sbuf[sl] + (c * N)
                pltpu.async_copy(h_ref.at[sbuf.at[pl.ds(0, KB)]], rbuf, semG)

            def finish(b, sbuf, dbuf, rbuf, ebuf, semDE, semG, semP):
                pltpu.make_async_copy(dst_ref.at[pl.ds(base + b * KB, KB)],
                                      dbuf, semDE).wait()
                pltpu.make_async_copy(e_ref.at[pl.ds(ebase + b * KB, KB)],
                                      ebuf, semDE).wait()
                pltpu.make_async_copy(h_ref.at[sbuf.at[pl.ds(0, KB)]], rbuf,
                                      semG).wait()
                relu_add(rbuf, ebuf)
                pltpu.async_copy(rbuf, agg_s.at[dbuf], semP, add=True)

            start(0, sA, dA, rA, eA, semSA, semDEA, semPA)
            mid(0, sA, rA, semSA, semGA)

            def pair(i, _):
                b0 = 2 * i
                start(b0 + 1, sB, dB, rB, eB, semSB, semDEB, semPB)
                mid(b0 + 1, sB, rB, semSB, semGB)
                finish(b0, sA, dA, rA, eA, semDEA, semGA, semPA)
                start(b0 + 2, sA, dA, rA, eA, semSA, semDEA, semPA)
                mid(b0 + 2, sA, rA, semSA, semGA)
                finish(b0 + 1, sB, dB, rB, eB, semDEB, semGB, semPB)
                return 0

            lax.fori_loop(0, NBATCH // 2 - 1, pair, 0)
            start(NBATCH - 1, sB, dB, rB, eB, semSB, semDEB, semPB)
            mid(NBATCH - 1, sB, rB, semSB, semGB)
            finish(NBATCH - 2, sA, dA, rA, eA, semDEA, semGA, semPA)
            finish(NBATCH - 1, sB, dB, rB, eB, semDEB, semGB, semPB)
            pltpu.make_async_copy(rA, agg_s.at[dA], semPA).wait()
            pltpu.make_async_copy(rB, agg_s.at[dB], semPB).wait()

            plsc.subcore_barrier()
            for j in range(RPT // KB):
                r0 = sid * RPT + j * KB
                pltpu.sync_copy(agg_s.at[pl.ds(r0, KB)],
                                out_ref.at[pl.ds(c * NP + r0, KB)])
            plsc.subcore_barrier()


def _sc_message_agg(h_flat, e_flat, src, dst):
    mesh = plsc.VectorSubcoreMesh(core_axis_name="c", subcore_axis_name="s")
    return pl.kernel(
        _sc_msg_body,
        out_type=jax.ShapeDtypeStruct((NCH * NP, HC), jnp.float32),
        mesh=mesh,
        scratch_types=[
            pltpu.MemorySpace.VMEM_SHARED((NP, HC), jnp.float32),
            pltpu.VMEM((KBP,), jnp.int32),
            pltpu.VMEM((KBP,), jnp.int32),
            pltpu.VMEM((KB,), jnp.int32),
            pltpu.VMEM((KB,), jnp.int32),
            pltpu.VMEM((KB, HC), jnp.float32),
            pltpu.VMEM((KB, HC), jnp.float32),
            pltpu.VMEM((KB, HC), jnp.float32),
            pltpu.VMEM((KB, HC), jnp.float32),
            pltpu.SemaphoreType.DMA,
            pltpu.SemaphoreType.DMA,
            pltpu.SemaphoreType.DMA,
            pltpu.SemaphoreType.DMA,
            pltpu.SemaphoreType.DMA,
            pltpu.SemaphoreType.DMA,
            pltpu.SemaphoreType.DMA,
            pltpu.SemaphoreType.DMA,
        ],
    )(h_flat, e_flat, src, dst)


def _enc_body(x_ref, w_ref, b_ref, out_ref):
    out_ref[0] = jnp.dot(x_ref[...], w_ref[...],
                         preferred_element_type=jnp.float32) + b_ref[...]


def _encoder(x, W_enc, b_enc):
    return pl.pallas_call(
        _enc_body,
        grid=(NCH, NRB),
        in_specs=[
            pl.BlockSpec((RB, F_IN), lambda c, r: (r, 0)),
            pl.BlockSpec((F_IN, HC), lambda c, r: (0, c)),
            pl.BlockSpec((1, HC), lambda c, r: (0, c)),
        ],
        out_specs=pl.BlockSpec((1, RB, HC), lambda c, r: (c, r, 0)),
        out_shape=jax.ShapeDtypeStruct((NCH, N, HC), jnp.float32),
    )(x, W_enc, b_enc.reshape(1, H))


def _edge_body(ea_ref, w_ref, b_ref, out_ref):
    out_ref[0] = jnp.dot(ea_ref[...], w_ref[...],
                         preferred_element_type=jnp.float32) + b_ref[...]


def _edge_mlp(edge_attr, We_l, be_l):
    return pl.pallas_call(
        _edge_body,
        grid=(NCH, NEB),
        in_specs=[
            pl.BlockSpec((EB, F_EDGE), lambda c, r: (r, 0)),
            pl.BlockSpec((F_EDGE, HC), lambda c, r: (0, c)),
            pl.BlockSpec((1, HC), lambda c, r: (0, c)),
        ],
        out_specs=pl.BlockSpec((1, EB, HC), lambda c, r: (c, r, 0)),
        out_shape=jax.ShapeDtypeStruct((NCH, E, HC), jnp.float32),
    )(edge_attr, We_l, be_l.reshape(1, H))


def _k1_body(h4_ref, a4_ref, eps_ref, w_ref, b_ref, y_ref, st_ref):
    r = pl.program_id(0)
    hcat = jnp.concatenate([h4_ref[c] for c in range(NCH)], axis=1)
    acat = jnp.concatenate([a4_ref[c] for c in range(NCH)], axis=1)
    z = (1.0 + eps_ref[0, 0]) * hcat + acat
    y = jnp.dot(z, w_ref[...], preferred_element_type=jnp.float32) + b_ref[...]
    y_ref[...] = y
    cs = jnp.sum(y, axis=0, keepdims=True)
    cq = jnp.sum(y * y, axis=0, keepdims=True)
    row = lax.broadcasted_iota(jnp.int32, (8, 2 * H), 0)
    upd = jnp.where(row == 0, cs, 0.0) + jnp.where(row == 1, cq, 0.0)

    @pl.when(r == 0)
    def _():
        st_ref[...] = jnp.zeros_like(st_ref)

    st_ref[...] += upd


def _k1(H4, A4, eps_l, W1_l, b1_l):
    return pl.pallas_call(
        _k1_body,
        grid=(NRB,),
        in_specs=[
            pl.BlockSpec((NCH, RB, HC), lambda r: (0, r, 0)),
            pl.BlockSpec((NCH, RB, HC), lambda r: (0, r, 0)),
            pl.BlockSpec(memory_space=pltpu.MemorySpace.SMEM),
            pl.BlockSpec((H, 2 * H), lambda r: (0, 0)),
            pl.BlockSpec((1, 2 * H), lambda r: (0, 0)),
        ],
        out_specs=[
            pl.BlockSpec((RB, 2 * H), lambda r: (r, 0)),
            pl.BlockSpec((8, 2 * H), lambda r: (0, 0)),
        ],
        out_shape=[
            jax.ShapeDtypeStruct((N, 2 * H), jnp.float32),
            jax.ShapeDtypeStruct((8, 2 * H), jnp.float32),
        ],
    )(H4, A4, eps_l.reshape(1, 1), W1_l, b1_l.reshape(1, 2 * H))


def _bn_coeffs(st, g, bt):
    mean = st[0:1, :] / N
    var = st[1:2, :] / N - mean * mean
    a = g * lax.rsqrt(var + 1e-5)
    b = bt - mean * a
    return a, b


def _k2_body(y_ref, st_ref, g_ref, bt_ref, w_ref, b_ref, y2_ref, st2_ref):
    r = pl.program_id(0)
    a, b = _bn_coeffs(st_ref[...], g_ref[...], bt_ref[...])
    t = jnp.maximum(y_ref[...] * a + b, 0.0)
    y2 = jnp.dot(t, w_ref[...], preferred_element_type=jnp.float32) + b_ref[...]
    y2_ref[...] = y2
    cs = jnp.sum(y2, axis=0, keepdims=True)
    cq = jnp.sum(y2 * y2, axis=0, keepdims=True)
    row = lax.broadcasted_iota(jnp.int32, (8, H), 0)
    upd = jnp.where(row == 0, cs, 0.0) + jnp.where(row == 1, cq, 0.0)

    @pl.when(r == 0)
    def _():
        st2_ref[...] = jnp.zeros_like(st2_ref)

    st2_ref[...] += upd


def _k2(y, st1, g1_l, bt1_l, W2_l, b2_l):
    return pl.pallas_call(
        _k2_body,
        grid=(NRB,),
        in_specs=[
            pl.BlockSpec((RB, 2 * H), lambda r: (r, 0)),
            pl.BlockSpec((8, 2 * H), lambda r: (0, 0)),
            pl.BlockSpec((1, 2 * H), lambda r: (0, 0)),
            pl.BlockSpec((1, 2 * H), lambda r: (0, 0)),
            pl.BlockSpec((2 * H, H), lambda r: (0, 0)),
            pl.BlockSpec((1, H), lambda r: (0, 0)),
        ],
        out_specs=[
            pl.BlockSpec((RB, H), lambda r: (r, 0)),
            pl.BlockSpec((8, H), lambda r: (0, 0)),
        ],
        out_shape=[
            jax.ShapeDtypeStruct((N, H), jnp.float32),
            jax.ShapeDtypeStruct((8, H), jnp.float32),
        ],
    )(y, st1, g1_l.reshape(1, 2 * H), bt1_l.reshape(1, 2 * H), W2_l,
      b2_l.reshape(1, H))


def _k3_body(y2_ref, st_ref, g_ref, bt_ref, h4_ref):
    a, b = _bn_coeffs(st_ref[...], g_ref[...], bt_ref[...])
    h4_ref[0] = jnp.maximum(y2_ref[...] * a + b, 0.0)


def _k3(y2, st2, g2_l, bt2_l):
    return pl.pallas_call(
        _k3_body,
        grid=(NCH, NRB),
        in_specs=[
            pl.BlockSpec((RB, HC), lambda c, r: (r, c)),
            pl.BlockSpec((8, HC), lambda c, r: (0, c)),
            pl.BlockSpec((1, HC), lambda c, r: (0, c)),
            pl.BlockSpec((1, HC), lambda c, r: (0, c)),
        ],
        out_specs=pl.BlockSpec((1, RB, HC), lambda c, r: (c, r, 0)),
        out_shape=jax.ShapeDtypeStruct((NCH, N, HC), jnp.float32),
    )(y2, st2, g2_l.reshape(1, H), bt2_l.reshape(1, H))


def _k3pool_body(y2_ref, st_ref, g_ref, bt_ref, bf_ref, sums_ref, cnt_ref):
    c = pl.program_id(0)
    r = pl.program_id(1)
    a, b = _bn_coeffs(st_ref[...], g_ref[...], bt_ref[...])
    h = jnp.maximum(y2_ref[...] * a + b, 0.0)
    gid = lax.broadcasted_iota(jnp.int32, (RB, G), 1)
    onehot = (bf_ref[...] == gid).astype(jnp.float32)
    part = lax.dot_general(onehot, h, (((0,), (0,)), ((), ())),
                           preferred_element_type=jnp.float32)

    @pl.when(r == 0)
    def _():
        sums_ref[...] = jnp.zeros_like(sums_ref)

    sums_ref[...] += part

    @pl.when(c == 0)
    def _():
        cpart = lax.dot_general(onehot, jnp.ones((RB, HC), jnp.float32),
                                (((0,), (0,)), ((), ())),
                                preferred_element_type=jnp.float32)

        @pl.when(r == 0)
        def _():
            cnt_ref[...] = jnp.zeros_like(cnt_ref)

        cnt_ref[...] += cpart


def _k3pool(y2, st2, g2_l, bt2_l, batch_f):
    return pl.pallas_call(
        _k3pool_body,
        grid=(NCH, NRB),
        in_specs=[
            pl.BlockSpec((RB, HC), lambda c, r: (r, c)),
            pl.BlockSpec((8, HC), lambda c, r: (0, c)),
            pl.BlockSpec((1, HC), lambda c, r: (0, c)),
            pl.BlockSpec((1, HC), lambda c, r: (0, c)),
            pl.BlockSpec((RB, 1), lambda c, r: (r, 0)),
        ],
        out_specs=[
            pl.BlockSpec((G, HC), lambda c, r: (0, c)),
            pl.BlockSpec((G, HC), lambda c, r: (0, 0)),
        ],
        out_shape=[
            jax.ShapeDtypeStruct((G, H), jnp.float32),
            jax.ShapeDtypeStruct((G, HC), jnp.float32),
        ],
    )(y2, st2, g2_l.reshape(1, H), bt2_l.reshape(1, H), batch_f)


def _gelu(x):
    return 0.5 * x * (1.0 + lax.erf(x * (2.0 ** -0.5)))


def _proj_body(sums_ref, cnt_ref, w1_ref, b1_ref, w2_ref, b2_ref, wb_ref,
               bb_ref, wo_ref, bo_ref, out_ref):
    emb = sums_ref[...] / jnp.maximum(cnt_ref[:, 0:1], 1.0)
    p = _gelu(jnp.dot(emb, w1_ref[...], preferred_element_type=jnp.float32)
              + b1_ref[...])
    p = _gelu(jnp.dot(p, w2_ref[...], preferred_element_type=jnp.float32)
              + b2_ref[...])
    p = _gelu(jnp.dot(p, wb_ref[...], preferred_element_type=jnp.float32)
              + bb_ref[...])
    p = jnp.dot(p, wo_ref[...], preferred_element_type=jnp.float32) + bo_ref[...]
    nrm = jnp.maximum(
        jnp.sqrt(jnp.sum(p * p, axis=1, keepdims=True)), 1e-12)
    out_ref[...] = p / nrm


def _projection(sums, cnt, Wp1, bp1, Wp2, bp2, Wpb, bpb, Wpo, bpo):
    return pl.pallas_call(
        _proj_body,
        out_shape=jax.ShapeDtypeStruct((G, PO), jnp.float32),
    )(sums, cnt, Wp1, bp1.reshape(1, PH), Wp2, bp2.reshape(1, PH),
      Wpb, bpb.reshape(1, PB), Wpo, bpo.reshape(1, PO))


def kernel(x, edge_index, edge_attr, batch, W_enc, b_enc, eps, We, be, W1, b1,
           g1, bt1, W2, b2, g2, bt2, Wp1, bp1, Wp2, bp2, Wpb, bpb, Wpo, bpo):
    src = edge_index[0]
    dst = edge_index[1]
    batch_i = batch.reshape(N, 1)

    H4 = _encoder(x, W_enc, b_enc)
    for l in range(3):
        e4 = _edge_mlp(edge_attr, We[l], be[l])
        agg = _sc_message_agg(H4.reshape(NCH * N, HC),
                              e4.reshape(NCH * E, HC), src, dst)
        y, st1 = _k1(H4, agg.reshape(NCH, NP, HC), eps[l], W1[l], b1[l])
        y2, st2 = _k2(y, st1, g1[l], bt1[l], W2[l], b2[l])
        if l < 2:
            H4 = _k3(y2, st2, g2[l], bt2[l])
        else:
            sums, cnt = _k3pool(y2, st2, g2[l], bt2[l], batch_i)
    return _projection(sums, cnt, Wp1, bp1, Wp2, bp2, Wpb, bpb, Wpo, bpo)


# speedup vs baseline: 2.4965x; 1.0127x over previous
import functools

import jax
import jax.numpy as jnp
from jax import lax
from jax.experimental import pallas as pl
from jax.experimental.pallas import tpu as pltpu
from jax.experimental.pallas import tpu_sc as plsc

N = 10000
E = 160000
F_IN = 256
F_EDGE = 16
H = 512
HC = 128
NCH = H // HC
G = 64
PH = 2048
PB = 256
PO = 256

RB = 2000
NRB = N // RB
EB = 8000
NEB = E // EB

SC_TILES = 16
EPT = E // SC_TILES
KB = 40
KBP = 48
NBATCH = EPT // KB
DEPTH = 3
NP = 10240
RPT = NP // SC_TILES


def _sc_msg_body(h_ref, e_ref, src_ref, dst_ref, out_ref,
                 agg_s, sA, sB, sC, dA, dB, dC, rA, rB, rC, eA, eB, eC,
                 semSA, semSB, semSC, semDEA, semDEB, semDEC,
                 semGA, semGB, semGC, semPA, semPB, semPC):
    cid = lax.axis_index("c")
    sid = lax.axis_index("s")
    sbufs, dbufs, rbufs, ebufs = (sA, sB, sC), (dA, dB, dC), (rA, rB, rC), \
        (eA, eB, eC)
    semSs, semDEs, semGs, semPs = (semSA, semSB, semSC), \
        (semDEA, semDEB, semDEC), (semGA, semGB, semGC), (semPA, semPB, semPC)

    def relu_add(rbuf, ebuf):
        @plsc.parallel_loop(0, KB, 1, unroll=2)
        def _(r):
            for j in range(HC // 16):
                sl = pl.ds(j * 16, 16)
                rbuf[r, sl] = jnp.maximum(rbuf[r, sl] + ebuf[r, sl], 0.0)

    zv = jnp.zeros((16,), jnp.float32)

    for c in range(NCH):
        @pl.when(cid == c // 2)
        def _(c=c):
            base = sid * EPT
            ebase = c * E + base

            def zfill(i, _):
                for j in range(HC // 16):
                    rA[i, pl.ds(j * 16, 16)] = zv
                return 0

            lax.fori_loop(0, KB, zfill, 0)
            for j in range(RPT // KB):
                pltpu.sync_copy(rA, agg_s.at[pl.ds(sid * RPT + j * KB, KB)])
            plsc.subcore_barrier()

            def start(b, sbuf, dbuf, rbuf, ebuf, semS, semDE, semP):
                @pl.when(b >= DEPTH)
                def _():
                    pltpu.make_async_copy(rbuf, agg_s.at[dbuf], semP).wait()

                pltpu.async_copy(src_ref.at[pl.ds(base + b * KB, KB)],
                                 sbuf.at[pl.ds(0, KB)], semS)
                pltpu.async_copy(dst_ref.at[pl.ds(base + b * KB, KB)], dbuf,
                                 semDE)
                pltpu.async_copy(e_ref.at[pl.ds(ebase + b * KB, KB)], ebuf,
                                 semDE)

            def mid(b, sbuf, rbuf, semS, semG):
                pltpu.make_async_copy(src_ref.at[pl.ds(base + b * KB, KB)],
                                      sbuf.at[pl.ds(0, KB)], semS).wait()
                if c > 0:
                    for j in range(KBP // 16):
                        sl = pl.ds(j * 16, 16)
                        sbuf[sl] = sbuf[sl] + (c * N)
                pltpu.async_copy(h_ref.at[sbuf.at[pl.ds(0, KB)]], rbuf, semG)

            def finish(b, sbuf, dbuf, rbuf, ebuf, semDE, semG, semP):
                pltpu.make_async_copy(dst_ref.at[pl.ds(base + b * KB, KB)],
                                      dbuf, semDE).wait()
                pltpu.make_async_copy(e_ref.at[pl.ds(ebase + b * KB, KB)],
                                      ebuf, semDE).wait()
                pltpu.make_async_copy(h_ref.at[sbuf.at[pl.ds(0, KB)]], rbuf,
                                      semG).wait()
                relu_add(rbuf, ebuf)
                pltpu.async_copy(rbuf, agg_s.at[dbuf], semP, add=True)

            def startk(b, k):
                start(b, sbufs[k], dbufs[k], rbufs[k], ebufs[k], semSs[k],
                      semDEs[k], semPs[k])

            def midk(b, k):
                mid(b, sbufs[k], rbufs[k], semSs[k], semGs[k])

            def finishk(b, k):
                finish(b, sbufs[k], dbufs[k], rbufs[k], ebufs[k], semDEs[k],
                       semGs[k], semPs[k])

            startk(0, 0)
            startk(1, 1)
            startk(2, 2)
            midk(0, 0)
            midk(1, 1)

            def triple(i, _):
                for k in range(DEPTH):
                    b = DEPTH * i + k
                    midk(b + 2, (k + 2) % DEPTH)
                    finishk(b, k)
                    startk(b + DEPTH, k)
                return 0

            NFULL = (NBATCH - DEPTH - 1) // DEPTH
            lax.fori_loop(0, NFULL, triple, 0)
            for b in range(DEPTH * NFULL, NBATCH):
                if b + 2 < NBATCH:
                    midk(b + 2, (b + 2) % DEPTH)
                finishk(b, b % DEPTH)
                if b + DEPTH < NBATCH:
                    startk(b + DEPTH, b % DEPTH)
            for k in range(DEPTH):
                bl = NBATCH - DEPTH + k
                pltpu.make_async_copy(rbufs[bl % DEPTH],
                                      agg_s.at[dbufs[bl % DEPTH]],
                                      semPs[bl % DEPTH]).wait()

            plsc.subcore_barrier()
            for j in range(RPT // KB):
                r0 = sid * RPT + j * KB
                pltpu.sync_copy(agg_s.at[pl.ds(r0, KB)],
                                out_ref.at[pl.ds(c * NP + r0, KB)])
            plsc.subcore_barrier()


def _sc_message_agg(h_flat, e_flat, src, dst):
    mesh = plsc.VectorSubcoreMesh(core_axis_name="c", subcore_axis_name="s")
    return pl.kernel(
        _sc_msg_body,
        out_type=jax.ShapeDtypeStruct((NCH * NP, HC), jnp.float32),
        mesh=mesh,
        scratch_types=(
            [pltpu.MemorySpace.VMEM_SHARED((NP, HC), jnp.float32)]
            + [pltpu.VMEM((KBP,), jnp.int32)] * DEPTH
            + [pltpu.VMEM((KB,), jnp.int32)] * DEPTH
            + [pltpu.VMEM((KB, HC), jnp.float32)] * DEPTH
            + [pltpu.VMEM((KB, HC), jnp.float32)] * DEPTH
            + [pltpu.SemaphoreType.DMA] * (4 * DEPTH)
        ),
    )(h_flat, e_flat, src, dst)


def _enc_body(x_ref, w_ref, b_ref, out_ref):
    out_ref[0] = jnp.dot(x_ref[...], w_ref[...],
                         preferred_element_type=jnp.float32) + b_ref[...]


def _encoder(x, W_enc, b_enc):
    return pl.pallas_call(
        _enc_body,
        grid=(NCH, NRB),
        in_specs=[
            pl.BlockSpec((RB, F_IN), lambda c, r: (r, 0)),
            pl.BlockSpec((F_IN, HC), lambda c, r: (0, c)),
            pl.BlockSpec((1, HC), lambda c, r: (0, c)),
        ],
        out_specs=pl.BlockSpec((1, RB, HC), lambda c, r: (c, r, 0)),
        out_shape=jax.ShapeDtypeStruct((NCH, N, HC), jnp.float32),
    )(x, W_enc, b_enc.reshape(1, H))


def _edge_body(ea_ref, w_ref, b_ref, out_ref):
    out_ref[0] = jnp.dot(ea_ref[...], w_ref[...],
                         preferred_element_type=jnp.float32) + b_ref[...]


def _edge_mlp(edge_attr, We_l, be_l):
    return pl.pallas_call(
        _edge_body,
        grid=(NCH, NEB),
        in_specs=[
            pl.BlockSpec((EB, F_EDGE), lambda c, r: (r, 0)),
            pl.BlockSpec((F_EDGE, HC), lambda c, r: (0, c)),
            pl.BlockSpec((1, HC), lambda c, r: (0, c)),
        ],
        out_specs=pl.BlockSpec((1, EB, HC), lambda c, r: (c, r, 0)),
        out_shape=jax.ShapeDtypeStruct((NCH, E, HC), jnp.float32),
    )(edge_attr, We_l, be_l.reshape(1, H))


def _k1_body(h4_ref, a4_ref, eps_ref, w_ref, b_ref, y_ref, st_ref):
    r = pl.program_id(0)
    hcat = jnp.concatenate([h4_ref[c] for c in range(NCH)], axis=1)
    acat = jnp.concatenate([a4_ref[c] for c in range(NCH)], axis=1)
    z = (1.0 + eps_ref[0, 0]) * hcat + acat
    y = jnp.dot(z, w_ref[...], preferred_element_type=jnp.float32) + b_ref[...]
    y_ref[...] = y
    cs = jnp.sum(y, axis=0, keepdims=True)
    cq = jnp.sum(y * y, axis=0, keepdims=True)
    row = lax.broadcasted_iota(jnp.int32, (8, 2 * H), 0)
    upd = jnp.where(row == 0, cs, 0.0) + jnp.where(row == 1, cq, 0.0)

    @pl.when(r == 0)
    def _():
        st_ref[...] = jnp.zeros_like(st_ref)

    st_ref[...] += upd


def _k1(H4, A4, eps_l, W1_l, b1_l):
    return pl.pallas_call(
        _k1_body,
        grid=(NRB,),
        in_specs=[
            pl.BlockSpec((NCH, RB, HC), lambda r: (0, r, 0)),
            pl.BlockSpec((NCH, RB, HC), lambda r: (0, r, 0)),
            pl.BlockSpec(memory_space=pltpu.MemorySpace.SMEM),
            pl.BlockSpec((H, 2 * H), lambda r: (0, 0)),
            pl.BlockSpec((1, 2 * H), lambda r: (0, 0)),
        ],
        out_specs=[
            pl.BlockSpec((RB, 2 * H), lambda r: (r, 0)),
            pl.BlockSpec((8, 2 * H), lambda r: (0, 0)),
        ],
        out_shape=[
            jax.ShapeDtypeStruct((N, 2 * H), jnp.float32),
            jax.ShapeDtypeStruct((8, 2 * H), jnp.float32),
        ],
    )(H4, A4, eps_l.reshape(1, 1), W1_l, b1_l.reshape(1, 2 * H))


def _bn_coeffs(st, g, bt):
    mean = st[0:1, :] / N
    var = st[1:2, :] / N - mean * mean
    a = g * lax.rsqrt(var + 1e-5)
    b = bt - mean * a
    return a, b


def _k2_body(y_ref, st_ref, g_ref, bt_ref, w_ref, b_ref, y2_ref, st2_ref):
    r = pl.program_id(0)
    a, b = _bn_coeffs(st_ref[...], g_ref[...], bt_ref[...])
    t = jnp.maximum(y_ref[...] * a + b, 0.0)
    y2 = jnp.dot(t, w_ref[...], preferred_element_type=jnp.float32) + b_ref[...]
    y2_ref[...] = y2
    cs = jnp.sum(y2, axis=0, keepdims=True)
    cq = jnp.sum(y2 * y2, axis=0, keepdims=True)
    row = lax.broadcasted_iota(jnp.int32, (8, H), 0)
    upd = jnp.where(row == 0, cs, 0.0) + jnp.where(row == 1, cq, 0.0)

    @pl.when(r == 0)
    def _():
        st2_ref[...] = jnp.zeros_like(st2_ref)

    st2_ref[...] += upd


def _k2(y, st1, g1_l, bt1_l, W2_l, b2_l):
    return pl.pallas_call(
        _k2_body,
        grid=(NRB,),
        in_specs=[
            pl.BlockSpec((RB, 2 * H), lambda r: (r, 0)),
            pl.BlockSpec((8, 2 * H), lambda r: (0, 0)),
            pl.BlockSpec((1, 2 * H), lambda r: (0, 0)),
            pl.BlockSpec((1, 2 * H), lambda r: (0, 0)),
            pl.BlockSpec((2 * H, H), lambda r: (0, 0)),
            pl.BlockSpec((1, H), lambda r: (0, 0)),
        ],
        out_specs=[
            pl.BlockSpec((RB, H), lambda r: (r, 0)),
            pl.BlockSpec((8, H), lambda r: (0, 0)),
        ],
        out_shape=[
            jax.ShapeDtypeStruct((N, H), jnp.float32),
            jax.ShapeDtypeStruct((8, H), jnp.float32),
        ],
    )(y, st1, g1_l.reshape(1, 2 * H), bt1_l.reshape(1, 2 * H), W2_l,
      b2_l.reshape(1, H))


def _k3_body(y2_ref, st_ref, g_ref, bt_ref, h4_ref):
    a, b = _bn_coeffs(st_ref[...], g_ref[...], bt_ref[...])
    h4_ref[0] = jnp.maximum(y2_ref[...] * a + b, 0.0)


def _k3(y2, st2, g2_l, bt2_l):
    return pl.pallas_call(
        _k3_body,
        grid=(NCH, NRB),
        in_specs=[
            pl.BlockSpec((RB, HC), lambda c, r: (r, c)),
            pl.BlockSpec((8, HC), lambda c, r: (0, c)),
            pl.BlockSpec((1, HC), lambda c, r: (0, c)),
            pl.BlockSpec((1, HC), lambda c, r: (0, c)),
        ],
        out_specs=pl.BlockSpec((1, RB, HC), lambda c, r: (c, r, 0)),
        out_shape=jax.ShapeDtypeStruct((NCH, N, HC), jnp.float32),
    )(y2, st2, g2_l.reshape(1, H), bt2_l.reshape(1, H))


def _k3pool_body(y2_ref, st_ref, g_ref, bt_ref, bf_ref, sums_ref, cnt_ref):
    c = pl.program_id(0)
    r = pl.program_id(1)
    a, b = _bn_coeffs(st_ref[...], g_ref[...], bt_ref[...])
    h = jnp.maximum(y2_ref[...] * a + b, 0.0)
    gid = lax.broadcasted_iota(jnp.int32, (RB, G), 1)
    onehot = (bf_ref[...] == gid).astype(jnp.float32)
    part = lax.dot_general(onehot, h, (((0,), (0,)), ((), ())),
                           preferred_element_type=jnp.float32)

    @pl.when(r == 0)
    def _():
        sums_ref[...] = jnp.zeros_like(sums_ref)

    sums_ref[...] += part

    @pl.when(c == 0)
    def _():
        cpart = lax.dot_general(onehot, jnp.ones((RB, HC), jnp.float32),
                                (((0,), (0,)), ((), ())),
                                preferred_element_type=jnp.float32)

        @pl.when(r == 0)
        def _():
            cnt_ref[...] = jnp.zeros_like(cnt_ref)

        cnt_ref[...] += cpart


def _k3pool(y2, st2, g2_l, bt2_l, batch_f):
    return pl.pallas_call(
        _k3pool_body,
        grid=(NCH, NRB),
        in_specs=[
            pl.BlockSpec((RB, HC), lambda c, r: (r, c)),
            pl.BlockSpec((8, HC), lambda c, r: (0, c)),
            pl.BlockSpec((1, HC), lambda c, r: (0, c)),
            pl.BlockSpec((1, HC), lambda c, r: (0, c)),
            pl.BlockSpec((RB, 1), lambda c, r: (r, 0)),
        ],
        out_specs=[
            pl.BlockSpec((G, HC), lambda c, r: (0, c)),
            pl.BlockSpec((G, HC), lambda c, r: (0, 0)),
        ],
        out_shape=[
            jax.ShapeDtypeStruct((G, H), jnp.float32),
            jax.ShapeDtypeStruct((G, HC), jnp.float32),
        ],
    )(y2, st2, g2_l.reshape(1, H), bt2_l.reshape(1, H), batch_f)


def _gelu(x):
    return 0.5 * x * (1.0 + lax.erf(x * (2.0 ** -0.5)))


def _proj_body(sums_ref, cnt_ref, w1_ref, b1_ref, w2_ref, b2_ref, wb_ref,
               bb_ref, wo_ref, bo_ref, out_ref):
    emb = sums_ref[...] / jnp.maximum(cnt_ref[:, 0:1], 1.0)
    p = _gelu(jnp.dot(emb, w1_ref[...], preferred_element_type=jnp.float32)
              + b1_ref[...])
    p = _gelu(jnp.dot(p, w2_ref[...], preferred_element_type=jnp.float32)
              + b2_ref[...])
    p = _gelu(jnp.dot(p, wb_ref[...], preferred_element_type=jnp.float32)
              + bb_ref[...])
    p = jnp.dot(p, wo_ref[...], preferred_element_type=jnp.float32) + bo_ref[...]
    nrm = jnp.maximum(
        jnp.sqrt(jnp.sum(p * p, axis=1, keepdims=True)), 1e-12)
    out_ref[...] = p / nrm


def _projection(sums, cnt, Wp1, bp1, Wp2, bp2, Wpb, bpb, Wpo, bpo):
    return pl.pallas_call(
        _proj_body,
        out_shape=jax.ShapeDtypeStruct((G, PO), jnp.float32),
    )(sums, cnt, Wp1, bp1.reshape(1, PH), Wp2, bp2.reshape(1, PH),
      Wpb, bpb.reshape(1, PB), Wpo, bpo.reshape(1, PO))


def kernel(x, edge_index, edge_attr, batch, W_enc, b_enc, eps, We, be, W1, b1,
           g1, bt1, W2, b2, g2, bt2, Wp1, bp1, Wp2, bp2, Wpb, bpb, Wpo, bpo):
    src = edge_index[0]
    dst = edge_index[1]
    batch_i = batch.reshape(N, 1)

    H4 = _encoder(x, W_enc, b_enc)
    for l in range(3):
        e4 = _edge_mlp(edge_attr, We[l], be[l])
        agg = _sc_message_agg(H4.reshape(NCH * N, HC),
                              e4.reshape(NCH * E, HC), src, dst)
        y, st1 = _k1(H4, agg.reshape(NCH, NP, HC), eps[l], W1[l], b1[l])
        y2, st2 = _k2(y, st1, g1[l], bt1[l], W2[l], b2[l])
        if l < 2:
            H4 = _k3(y2, st2, g2[l], bt2[l])
        else:
            sums, cnt = _k3pool(y2, st2, g2[l], bt2[l], batch_i)
    return _projection(sums, cnt, Wp1, bp1, Wp2, bp2, Wpb, bpb, Wpo, bpo)
